# baseline (speedup 1.0000x reference)
import functools

import jax
import jax.numpy as jnp
from jax.experimental import pallas as pl
from jax.experimental.pallas import tpu as pltpu

N_NODES = 10000
N_EDGES = 160000
BE = 2000
BN = 2000


def _full(shape):
    return pl.BlockSpec(shape, lambda i: tuple(0 for _ in shape))


def _rows(block, ncols):
    return pl.BlockSpec((block, ncols), lambda i: (i, 0))


def _node_proj_body(x_ref, wi_ref, wj_ref, pd_ref, ps_ref):
    pd_ref[...] = jnp.dot(x_ref[...], wi_ref[...],
                          preferred_element_type=jnp.float32)
    ps_ref[...] = jnp.dot(x_ref[...], wj_ref[...],
                          preferred_element_type=jnp.float32)


def _node_proj(x, wi, wj):
    n, k = x.shape
    grid = n // BN
    return pl.pallas_call(
        _node_proj_body,
        grid=(grid,),
        in_specs=[_rows(BN, k), _full((k, 64)), _full((k, 64))],
        out_specs=[_rows(BN, 64), _rows(BN, 64)],
        out_shape=[
            jax.ShapeDtypeStruct((n, 64), jnp.float32),
            jax.ShapeDtypeStruct((n, 64), jnp.float32),
        ],
    )(x, wi, wj)


def _edge_mlp_body(t_ref, ea_ref, we_ref, b1_ref, w2_ref, b2_ref, w3_ref,
                   b3_ref, out_ref):
    u = (t_ref[...]
         + jnp.dot(ea_ref[...], we_ref[...], preferred_element_type=jnp.float32)
         + b1_ref[...])
    u = jnp.maximum(u, 0.0)
    u = jnp.dot(u, w2_ref[...], preferred_element_type=jnp.float32) + b2_ref[...]
    u = jnp.maximum(u, 0.0)
    out_ref[...] = (jnp.dot(u, w3_ref[...], preferred_element_type=jnp.float32)
                    + b3_ref[...])


def _edge_mlp(t, ea, we, b1, w2, b2, w3, b3):
    grid = N_EDGES // BE
    return pl.pallas_call(
        _edge_mlp_body,
        grid=(grid,),
        in_specs=[
            _rows(BE, 64), _rows(BE, 16),
            _full((16, 64)), _full((1, 64)),
            _full((64, 64)), _full((1, 64)),
            _full((64, 64)), _full((1, 64)),
        ],
        out_specs=_rows(BE, 64),
        out_shape=jax.ShapeDtypeStruct((N_EDGES, 64), jnp.float32),
    )(t, ea, we, b1.reshape(1, 64), w2, b2.reshape(1, 64), w3,
      b3.reshape(1, 64))


def _gamma_body(agg_ref, x_ref, wa_ref, wx_ref, b1_ref, w2_ref, b2_ref,
                w3_ref, b3_ref, out_ref, *, relu_out):
    u = (jnp.dot(agg_ref[...], wa_ref[...], preferred_element_type=jnp.float32)
         + jnp.dot(x_ref[...], wx_ref[...], preferred_element_type=jnp.float32)
         + b1_ref[...])
    u = jnp.maximum(u, 0.0)
    u = jnp.dot(u, w2_ref[...], preferred_element_type=jnp.float32) + b2_ref[...]
    u = jnp.maximum(u, 0.0)
    u = jnp.dot(u, w3_ref[...], preferred_element_type=jnp.float32) + b3_ref[...]
    if relu_out:
        u = jnp.maximum(u, 0.0)
    out_ref[...] = u


def _gamma(agg, x, wa, wx, b1, w2, b2, w3, b3, relu_out):
    n, k = x.shape
    grid = n // BN
    body = functools.partial(_gamma_body, relu_out=relu_out)
    return pl.pallas_call(
        body,
        grid=(grid,),
        in_specs=[
            _rows(BN, 64), _rows(BN, k),
            _full((64, 64)), _full((k, 64)), _full((1, 64)),
            _full((64, 64)), _full((1, 64)),
            _full((64, 64)), _full((1, 64)),
        ],
        out_specs=_rows(BN, 64),
        out_shape=jax.ShapeDtypeStruct((n, 64), jnp.float32),
    )(agg, x, wa, wx, b1.reshape(1, 64), w2, b2.reshape(1, 64), w3,
      b3.reshape(1, 64))


def _head_body(h_ref, w1_ref, b1_ref, w2_ref, b2_ref, w3_ref, b3_ref, out_ref):
    u = jnp.dot(h_ref[...], w1_ref[...], preferred_element_type=jnp.float32) + b1_ref[...]
    u = jnp.maximum(u, 0.0)
    u = jnp.dot(u, w2_ref[...], preferred_element_type=jnp.float32) + b2_ref[...]
    u = jnp.maximum(u, 0.0)
    out_ref[...] = jnp.dot(u, w3_ref[...], preferred_element_type=jnp.float32) + b3_ref[...]


def _head(h, params):
    (w1, b1), (w2, b2), (w3, b3) = params
    grid = h.shape[0] // BN
    return pl.pallas_call(
        _head_body,
        grid=(grid,),
        in_specs=[
            _rows(BN, 64),
            _full((64, 64)), _full((1, 64)),
            _full((64, 64)), _full((1, 64)),
            _full((64, 16)), _full((1, 16)),
        ],
        out_specs=_rows(BN, 16),
        out_shape=jax.ShapeDtypeStruct((h.shape[0], 16), jnp.float32),
    )(h, w1, b1.reshape(1, 64), w2, b2.reshape(1, 64), w3, b3.reshape(1, 16))


def _layer(x, edge_attr, src, dst, phi, gamma, relu_out):
    (wp1, bp1), (wp2, bp2), (wp3, bp3) = phi
    (wg1, bg1), (wg2, bg2), (wg3, bg3) = gamma
    k = x.shape[1]
    wi = wp1[:k]
    wj = wp1[k:2 * k]
    we = wp1[2 * k:]
    pd, ps = _node_proj(x, wi, wj)

    t = pd[dst] + ps[src]
    m3 = _edge_mlp(t, edge_attr, we, bp1, wp2, bp2, wp3, bp3)

    agg = jax.ops.segment_max(m3, dst, num_segments=N_NODES)
    agg = jnp.where(jnp.isneginf(agg), 0.0, agg)

    wa = wg1[:64]
    wx = wg1[64:]
    return _gamma(agg, x, wa, wx, bg1, wg2, bg2, wg3, bg3, relu_out)


def kernel(x, edge_attr, edge_index, params):
    phi1, gamma1, phi2, gamma2, head = params
    src = edge_index[0].astype(jnp.int32)
    dst = edge_index[1].astype(jnp.int32)
    h = _layer(x, edge_attr, src, dst, phi1, gamma1, relu_out=True)
    h = _layer(h, edge_attr, src, dst, phi2, gamma2, relu_out=False)
    return _head(h, head)
